# baseline (speedup 1.0000x reference)
import jax
import jax.numpy as jnp
from jax import lax
from jax.experimental import pallas as pl
from jax.experimental.pallas import tpu as pltpu
from jax.experimental.pallas import tpu_sc as plsc

_N_WORDS = 77
_D = 768
_BATCH = 1024
_TOTAL = _BATCH * _N_WORDS
_NC = 2
_NS = 16
_L = 16
_NW = _NC * _NS
_ROWS_PER_W = _TOTAL // _NW
_C = 8
_NCHUNK = _ROWS_PER_W // _C
_NSLOT = 4

_mesh = plsc.VectorSubcoreMesh(
    core_axis_name="c", subcore_axis_name="s", num_cores=_NC, num_subcores=_NS)

_SCRATCH = (
    [pltpu.VMEM((_ROWS_PER_W + _L,), jnp.int32), pltpu.VMEM((_ROWS_PER_W + _L,), jnp.int32)]
    + [pltpu.VMEM((_C, _D), jnp.float32)] * _NSLOT
    + [pltpu.VMEM((_N_WORDS, _D), jnp.float32)]
    + [pltpu.SemaphoreType.DMA] * (2 * _NSLOT)
)


def _emb_body(ids_hbm, pids_hbm, tok_hbm, pos_hbm, out_hbm,
              idx_at, idx_ap, *rest):
    rows_t = rest[0:_NSLOT]
    pos_v = rest[_NSLOT]
    gsem = rest[_NSLOT + 1:2 * _NSLOT + 1]
    ssem = rest[2 * _NSLOT + 1:3 * _NSLOT + 1]

    wid = lax.axis_index("s") * _NC + lax.axis_index("c")
    base = wid * _ROWS_PER_W

    def fire_gather(b, loc):
        pltpu.async_copy(tok_hbm.at[idx_at.at[pl.ds(loc, _C)]], rows_t[b], gsem[b])

    def wait_gather(b):
        pltpu.make_async_copy(tok_hbm.at[idx_at.at[pl.ds(0, _C)]], rows_t[b], gsem[b]).wait()

    def wait_store(b, off):
        pltpu.make_async_copy(rows_t[b], out_hbm.at[pl.ds(off, _C)], ssem[b]).wait()

    pltpu.sync_copy(ids_hbm.at[pl.ds(base, _ROWS_PER_W)], idx_at.at[pl.ds(0, _ROWS_PER_W)])
    pltpu.sync_copy(pids_hbm.at[pl.ds(base, _ROWS_PER_W)], idx_ap.at[pl.ds(0, _ROWS_PER_W)])
    pltpu.sync_copy(pos_hbm, pos_v)
    for s in range(_NSLOT - 1):
        fire_gather(s, s * _C)

    iota = lax.iota(jnp.int32, _L)

    @pl.loop(0, _NCHUNK, step=_NSLOT)
    def _grp(ci):
        for b in range(_NSLOT):
            c = ci + b
            off = base + c * _C
            wait_gather(b)

            pid_v = idx_ap[pl.ds(c * _C, _L)]

            @plsc.parallel_loop(0, _C, 1, unroll=6)
            def _row(r):
                rsplat = jnp.full((_L,), r, dtype=jnp.int32)
                pid_splat = lax.gather(
                    pid_v, rsplat[:, None],
                    dimension_numbers=lax.GatherDimensionNumbers(
                        offset_dims=(), collapsed_slice_dims=(0,),
                        start_index_map=(0,)),
                    slice_sizes=(1,),
                    mode=lax.GatherScatterMode.PROMISE_IN_BOUNDS)
                for j in range(_D // _L):
                    sl = pl.ds(j * _L, _L)
                    p = plsc.load_gather(pos_v, [pid_splat, iota + (j * _L)])
                    plsc.addupdate(rows_t[b].at[r, sl], p)

            pltpu.async_copy(rows_t[b], out_hbm.at[pl.ds(off, _C)], ssem[b])

            o = (b + _NSLOT - 1) % _NSLOT

            @pl.when(c >= 1)
            def _():
                wait_store(o, base)

            @pl.when(c + _NSLOT - 1 < _NCHUNK)
            def _():
                fire_gather(o, (c + _NSLOT - 1) * _C)

    wait_store((_NCHUNK - 1) % _NSLOT, base)


_emb_kernel = pl.kernel(
    _emb_body,
    out_type=jax.ShapeDtypeStruct((_TOTAL, _D), jnp.float32),
    mesh=_mesh,
    scratch_types=_SCRATCH,
    compiler_params=pltpu.CompilerParams(needs_layout_passes=False),
)


def kernel(input_ids, pos_ids, token_table, pos_table):
    ids = input_ids.astype(jnp.int32).T.reshape(-1)
    pids = pos_ids.astype(jnp.int32).T.reshape(-1)
    out = _emb_kernel(ids, pids, token_table, pos_table)
    return out.reshape(_N_WORDS, _BATCH, _D).transpose(1, 0, 2)


# speedup vs baseline: 2.4023x; 2.4023x over previous
import jax
import jax.numpy as jnp
from jax import lax
from jax.experimental import pallas as pl
from jax.experimental.pallas import tpu as pltpu
from jax.experimental.pallas import tpu_sc as plsc

_N_WORDS = 77
_D = 768
_BATCH = 1024
_TOTAL = _BATCH * _N_WORDS
_NC = 2
_NS = 16
_L = 16
_NW = _NC * _NS
_ROWS_PER_W = _TOTAL // _NW
_C = 8
_NCHUNK = _ROWS_PER_W // _C
_NSLOT = 4

_mesh = plsc.VectorSubcoreMesh(
    core_axis_name="c", subcore_axis_name="s", num_cores=_NC, num_subcores=_NS)

_SCRATCH = (
    [pltpu.VMEM((_ROWS_PER_W + _L,), jnp.int32), pltpu.VMEM((_ROWS_PER_W + _L,), jnp.int32)]
    + [pltpu.VMEM((_C, _D), jnp.float32)] * _NSLOT
    + [pltpu.VMEM((_N_WORDS, _D // 2), jnp.int32)]
    + [pltpu.SemaphoreType.DMA] * (2 * _NSLOT)
)


def _emb_body(ids_hbm, pids_hbm, tok_hbm, pos_hbm, out_hbm,
              idx_at, idx_ap, *rest):
    rows_t = rest[0:_NSLOT]
    pos_v = rest[_NSLOT]
    gsem = rest[_NSLOT + 1:2 * _NSLOT + 1]
    ssem = rest[2 * _NSLOT + 1:3 * _NSLOT + 1]

    wid = lax.axis_index("s") * _NC + lax.axis_index("c")
    base = wid * _ROWS_PER_W

    def fire_gather(b, loc):
        pltpu.async_copy(tok_hbm.at[idx_at.at[pl.ds(loc, _C)]], rows_t[b], gsem[b])

    def wait_gather(b):
        pltpu.make_async_copy(tok_hbm.at[idx_at.at[pl.ds(0, _C)]], rows_t[b], gsem[b]).wait()

    def wait_store(b, off):
        pltpu.make_async_copy(rows_t[b], out_hbm.at[pl.ds(off, _C)], ssem[b]).wait()

    pltpu.sync_copy(ids_hbm.at[pl.ds(base, _ROWS_PER_W)], idx_at.at[pl.ds(0, _ROWS_PER_W)])
    pltpu.sync_copy(pids_hbm.at[pl.ds(base, _ROWS_PER_W)], idx_ap.at[pl.ds(0, _ROWS_PER_W)])
    pltpu.sync_copy(pos_hbm, pos_v)
    for s in range(_NSLOT - 1):
        fire_gather(s, s * _C)

    iota = lax.iota(jnp.int32, _L)

    @pl.loop(0, _NCHUNK, step=_NSLOT)
    def _grp(ci):
        for b in range(_NSLOT):
            c = ci + b
            off = base + c * _C
            wait_gather(b)

            pid_v = idx_ap[pl.ds(c * _C, _L)]

            @plsc.parallel_loop(0, _C, 1, unroll=4)
            def _row(r):
                rsplat = jnp.full((_L,), r, dtype=jnp.int32)
                pid_splat = lax.gather(
                    pid_v, rsplat[:, None],
                    dimension_numbers=lax.GatherDimensionNumbers(
                        offset_dims=(), collapsed_slice_dims=(0,),
                        start_index_map=(0,)),
                    slice_sizes=(1,),
                    mode=lax.GatherScatterMode.PROMISE_IN_BOUNDS)
                for k in range(_D // (2 * _L)):
                    w = plsc.load_gather(pos_v, [pid_splat, iota + (k * _L)])
                    bf = plsc.bitcast(w, jnp.bfloat16)
                    lo, hi = plsc.unpack(
                        bf, format=plsc.PackFormat.INTERLEAVED,
                        preferred_element_type=jnp.float32)
                    plsc.addupdate(rows_t[b].at[r, pl.ds(2 * k * _L, _L)], lo)
                    plsc.addupdate(rows_t[b].at[r, pl.ds(2 * k * _L + _L, _L)], hi)

            pltpu.async_copy(rows_t[b], out_hbm.at[pl.ds(off, _C)], ssem[b])

            o = (b + _NSLOT - 1) % _NSLOT

            @pl.when(c >= 1)
            def _():
                wait_store(o, base)

            @pl.when(c + _NSLOT - 1 < _NCHUNK)
            def _():
                fire_gather(o, (c + _NSLOT - 1) * _C)

    wait_store((_NCHUNK - 1) % _NSLOT, base)


_emb_kernel = pl.kernel(
    _emb_body,
    out_type=jax.ShapeDtypeStruct((_TOTAL, _D), jnp.float32),
    mesh=_mesh,
    scratch_types=_SCRATCH,
    compiler_params=pltpu.CompilerParams(needs_layout_passes=False),
)


def kernel(input_ids, pos_ids, token_table, pos_table):
    ids = input_ids.astype(jnp.int32).T.reshape(-1)
    pids = pos_ids.astype(jnp.int32).T.reshape(-1)
    posr = pos_table.astype(jnp.bfloat16).reshape(_N_WORDS, _D // 32, 2, _L)
    pairs = jnp.stack([posr[:, :, 0, :], posr[:, :, 1, :]], axis=-1)
    pos_pack = lax.bitcast_convert_type(pairs, jnp.int32).reshape(_N_WORDS, _D // 2)
    out = _emb_kernel(ids, pids, token_table, pos_pack)
    return out.reshape(_N_WORDS, _BATCH, _D).transpose(1, 0, 2)
